# baseline (speedup 1.0000x reference)
import functools

import jax
import jax.numpy as jnp
from jax import lax
from jax.experimental import pallas as pl
from jax.experimental.pallas import tpu as pltpu
from jax.experimental.pallas import tpu_sc as plsc

B = 16384
EMB = 32
N_TAB = 4
HID = 256
OUT = 20
SCENE = 20

NC, NS = 2, 16
NW = NC * NS
BPW = B // NW
CHUNK = 128
JPT = BPW // CHUNK


TAB_ROWS = 86 + 86 + 24 + 8


def _sc_gather(ids_r, ctab):
    mesh = plsc.VectorSubcoreMesh(core_axis_name="c", subcore_axis_name="s")

    @functools.partial(
        pl.kernel,
        mesh=mesh,
        compiler_params=pltpu.CompilerParams(use_tc_tiling_on_sc=False, skip_device_barrier=True),
        out_type=jax.ShapeDtypeStruct((B, N_TAB * EMB), jnp.float32),
        scratch_types=[
            pltpu.VMEM((N_TAB, JPT, CHUNK), jnp.int32),
            pltpu.VMEM((N_TAB * BPW, EMB), jnp.float32),
            pltpu.VMEM_SHARED((TAB_ROWS, EMB), jnp.float32),
            pltpu.SemaphoreType.DMA,
            pltpu.SemaphoreType.DMA,
            pltpu.SemaphoreType.DMA,
        ],
    )
    def k(ids_h, ctab_h, out_h, idx_v, rows_v, ctab_s, sem_i, sem_g, sem_o):
        sid_ax = lax.axis_index("s")
        wid = sid_ax * NC + lax.axis_index("c")
        base = wid * BPW
        cps = [
            pltpu.async_copy(ids_h.at[t, wid], idx_v.at[t], sem_i)
            for t in range(N_TAB)
        ]
        @pl.when(sid_ax == 0)
        def _stage_tables():
            pltpu.sync_copy(ctab_h, ctab_s)
        for cp in cps:
            cp.wait()
        plsc.subcore_barrier()
        cps = [
            pltpu.async_copy(
                ctab_s.at[idx_v.at[t, j]],
                rows_v.at[pl.ds(BPW * t + CHUNK * j, CHUNK)],
                sem_g,
            )
            for t in range(N_TAB)
            for j in range(JPT)
        ]
        for cp in cps:
            cp.wait()
        cps = [
            pltpu.async_copy(
                rows_v.at[pl.ds(BPW * t, BPW)],
                out_h.at[pl.ds(base, BPW), pl.ds(EMB * t, EMB)],
                sem_o,
            )
            for t in range(N_TAB)
        ]
        for cp in cps:
            cp.wait()

    return k(ids_r, ctab)


def _dot(a, b, dims):
    return lax.dot_general(a, b, dimension_numbers=(dims, ((), ())),
                           preferred_element_type=jnp.float32)


def _mlp_body(g_ref, scT_ref, w1T_ref, b1_ref, w2T_ref, b2_ref, o_ref):
    hT = _dot(w1T_ref[:, pl.ds(0, N_TAB * EMB)], g_ref[...], ((1,), (1,)))
    hT += _dot(w1T_ref[:, pl.ds(N_TAB * EMB, SCENE)], scT_ref[...], ((1,), (0,)))
    hT = jnp.maximum(hT + b1_ref[...], 0.0)
    o_ref[...] = _dot(w2T_ref[...], hT, ((1,), (0,))) + b2_ref[...]


def _mlp_t(g, sceneT, w1T, b1c, w2T, b2c, blk=8192):
    grid = B // blk
    outT = pl.pallas_call(
        _mlp_body,
        grid=(grid,),
        in_specs=[
            pl.BlockSpec((blk, N_TAB * EMB), lambda i: (i, 0)),
            pl.BlockSpec((SCENE, blk), lambda i: (0, i)),
            pl.BlockSpec((HID, N_TAB * EMB + SCENE), lambda i: (0, 0)),
            pl.BlockSpec((HID, 1), lambda i: (0, 0)),
            pl.BlockSpec((OUT, HID), lambda i: (0, 0)),
            pl.BlockSpec((OUT, 1), lambda i: (0, 0)),
        ],
        out_specs=pl.BlockSpec((OUT, blk), lambda i: (0, i)),
        out_shape=jax.ShapeDtypeStruct((OUT, B), jnp.float32),
    )(g, sceneT, w1T, b1c, w2T, b2c)
    return outT


def kernel(killer_id, victim_id, move_id, stage_id, scene_tags,
           killer_table, victim_table, move_table, stage_table,
           W1, b1, W2, b2):
    ids_r = jnp.concatenate(
        [killer_id, victim_id + 86, move_id + 172, stage_id + 196]
    ).reshape(N_TAB, NW, JPT, CHUNK)
    ctab = jnp.concatenate(
        [killer_table, victim_table, move_table, stage_table], axis=0
    )
    g = _sc_gather(ids_r, ctab)
    outT = _mlp_t(g, scene_tags.T, W1.T, b1.reshape(HID, 1),
                  W2.T, b2.reshape(OUT, 1))
    return outT.T


# speedup vs baseline: 1.0011x; 1.0011x over previous
import functools

import jax
import jax.numpy as jnp
from jax import lax
from jax.experimental import pallas as pl
from jax.experimental.pallas import tpu as pltpu
from jax.experimental.pallas import tpu_sc as plsc

B = 16384
EMB = 32
N_TAB = 4
HID = 256
OUT = 20
SCENE = 20

NC, NS = 2, 16
NW = NC * NS
BPW = B // NW
CHUNK = 128
JPT = BPW // CHUNK


TAB_ROWS = 86 + 86 + 24 + 8


def _sc_gather(ids_r, ctab):
    mesh = plsc.VectorSubcoreMesh(core_axis_name="c", subcore_axis_name="s")

    @functools.partial(
        pl.kernel,
        mesh=mesh,
        compiler_params=pltpu.CompilerParams(use_tc_tiling_on_sc=False),
        out_type=jax.ShapeDtypeStruct((B, N_TAB * EMB), jnp.float32),
        scratch_types=[
            pltpu.VMEM((N_TAB, JPT, CHUNK), jnp.int32),
            pltpu.VMEM((N_TAB * BPW, EMB), jnp.float32),
            pltpu.VMEM_SHARED((TAB_ROWS, EMB), jnp.float32),
            pltpu.SemaphoreType.DMA,
            pltpu.SemaphoreType.DMA,
            pltpu.SemaphoreType.DMA,
        ],
    )
    def k(ids_h, ctab_h, out_h, idx_v, rows_v, ctab_s, sem_i, sem_g, sem_o):
        sid_ax = lax.axis_index("s")
        wid = sid_ax * NC + lax.axis_index("c")
        base = wid * BPW
        cps = [
            pltpu.async_copy(ids_h.at[t, wid], idx_v.at[t], sem_i)
            for t in range(N_TAB)
        ]
        @pl.when(sid_ax == 0)
        def _stage_tables():
            pltpu.sync_copy(ctab_h, ctab_s)
        for cp in cps:
            cp.wait()
        plsc.subcore_barrier()
        cps = [
            pltpu.async_copy(
                ctab_s.at[idx_v.at[t, j]],
                rows_v.at[pl.ds(BPW * t + CHUNK * j, CHUNK)],
                sem_g,
            )
            for t in range(N_TAB)
            for j in range(JPT)
        ]
        for cp in cps:
            cp.wait()
        cps = [
            pltpu.async_copy(
                rows_v.at[pl.ds(BPW * t, BPW)],
                out_h.at[pl.ds(base, BPW), pl.ds(EMB * t, EMB)],
                sem_o,
            )
            for t in range(N_TAB)
        ]
        for cp in cps:
            cp.wait()

    return k(ids_r, ctab)


def _dot(a, b, dims):
    return lax.dot_general(a, b, dimension_numbers=(dims, ((), ())),
                           preferred_element_type=jnp.float32)


def _mlp_body(g_ref, scT_ref, w1T_ref, b1_ref, w2T_ref, b2_ref, o_ref):
    hT = _dot(w1T_ref[:, pl.ds(0, N_TAB * EMB)], g_ref[...], ((1,), (1,)))
    hT += _dot(w1T_ref[:, pl.ds(N_TAB * EMB, SCENE)], scT_ref[...], ((1,), (0,)))
    hT = jnp.maximum(hT + b1_ref[...], 0.0)
    o_ref[...] = _dot(w2T_ref[...], hT, ((1,), (0,))) + b2_ref[...]


def _mlp_t(g, sceneT, w1T, b1c, w2T, b2c, blk=4096):
    grid = B // blk
    outT = pl.pallas_call(
        _mlp_body,
        grid=(grid,),
        in_specs=[
            pl.BlockSpec((blk, N_TAB * EMB), lambda i: (i, 0)),
            pl.BlockSpec((SCENE, blk), lambda i: (0, i)),
            pl.BlockSpec((HID, N_TAB * EMB + SCENE), lambda i: (0, 0)),
            pl.BlockSpec((HID, 1), lambda i: (0, 0)),
            pl.BlockSpec((OUT, HID), lambda i: (0, 0)),
            pl.BlockSpec((OUT, 1), lambda i: (0, 0)),
        ],
        out_specs=pl.BlockSpec((OUT, blk), lambda i: (0, i)),
        out_shape=jax.ShapeDtypeStruct((OUT, B), jnp.float32),
    )(g, sceneT, w1T, b1c, w2T, b2c)
    return outT


def kernel(killer_id, victim_id, move_id, stage_id, scene_tags,
           killer_table, victim_table, move_table, stage_table,
           W1, b1, W2, b2):
    ids_r = jnp.concatenate(
        [killer_id, victim_id + 86, move_id + 172, stage_id + 196]
    ).reshape(N_TAB, NW, JPT, CHUNK)
    ctab = jnp.concatenate(
        [killer_table, victim_table, move_table, stage_table], axis=0
    )
    g = _sc_gather(ids_r, ctab)
    outT = _mlp_t(g, scene_tags.T, W1.T, b1.reshape(HID, 1),
                  W2.T, b2.reshape(OUT, 1))
    return outT.T


# speedup vs baseline: 1.1230x; 1.1218x over previous
import functools

import jax
import jax.numpy as jnp
from jax import lax
from jax.experimental import pallas as pl
from jax.experimental.pallas import tpu as pltpu
from jax.experimental.pallas import tpu_sc as plsc

B = 16384
EMB = 32
N_TAB = 4
HID = 256
OUT = 20
SCENE = 20

NC, NS = 2, 16
NW = NC * NS
BPW = B // NW
CHUNK = 128
JPT = BPW // CHUNK


TAB_ROWS = 86 + 86 + 24 + 8
TAB_OFFS = (0, 86, 172, 196)


def _sc_gather(kid_r, vid_r, mid_r, sid_r, ctab):
    mesh = plsc.VectorSubcoreMesh(core_axis_name="c", subcore_axis_name="s")

    @functools.partial(
        pl.kernel,
        mesh=mesh,
        compiler_params=pltpu.CompilerParams(use_tc_tiling_on_sc=False),
        out_type=jax.ShapeDtypeStruct((B, N_TAB * EMB), jnp.float32),
        scratch_types=[
            pltpu.VMEM((N_TAB, JPT, CHUNK), jnp.int32),
            pltpu.VMEM((N_TAB * BPW, EMB), jnp.float32),
            pltpu.VMEM_SHARED((TAB_ROWS, EMB), jnp.float32),
            pltpu.SemaphoreType.DMA,
            pltpu.SemaphoreType.DMA,
            pltpu.SemaphoreType.DMA,
        ],
    )
    def k(kid_h, vid_h, mid_h, sid_h, ctab_h, out_h,
          idx_v, rows_v, ctab_s, sem_i, sem_g, sem_o):
        sid_ax = lax.axis_index("s")
        wid = sid_ax * NC + lax.axis_index("c")
        base = wid * BPW
        ids = (kid_h, vid_h, mid_h, sid_h)
        cps = [
            pltpu.async_copy(ids[t].at[wid], idx_v.at[t], sem_i)
            for t in range(N_TAB)
        ]
        @pl.when(sid_ax == 0)
        def _stage_tables():
            pltpu.sync_copy(ctab_h, ctab_s)
        for cp in cps:
            cp.wait()
        for t in range(1, N_TAB):
            for j in range(JPT):
                for m in range(CHUNK // 16):
                    sl = (t, j, pl.ds(16 * m, 16))
                    idx_v[sl] = idx_v[sl] + TAB_OFFS[t]
        plsc.subcore_barrier()
        cps = [
            pltpu.async_copy(
                ctab_s.at[idx_v.at[t, j]],
                rows_v.at[pl.ds(BPW * t + CHUNK * j, CHUNK)],
                sem_g,
            )
            for t in range(N_TAB)
            for j in range(JPT)
        ]
        for cp in cps:
            cp.wait()
        cps = [
            pltpu.async_copy(
                rows_v.at[pl.ds(BPW * t, BPW)],
                out_h.at[pl.ds(base, BPW), pl.ds(EMB * t, EMB)],
                sem_o,
            )
            for t in range(N_TAB)
        ]
        for cp in cps:
            cp.wait()

    return k(kid_r, vid_r, mid_r, sid_r, ctab)


def _dot(a, b, dims):
    return lax.dot_general(a, b, dimension_numbers=(dims, ((), ())),
                           preferred_element_type=jnp.float32)


def _mlp_body(g_ref, scT_ref, w1T_ref, b1_ref, w2T_ref, b2_ref, o_ref):
    hT = _dot(w1T_ref[:, pl.ds(0, N_TAB * EMB)], g_ref[...], ((1,), (1,)))
    hT += _dot(w1T_ref[:, pl.ds(N_TAB * EMB, SCENE)], scT_ref[...], ((1,), (0,)))
    hT = jnp.maximum(hT + b1_ref[...], 0.0)
    o_ref[...] = _dot(w2T_ref[...], hT, ((1,), (0,))) + b2_ref[...]


def _mlp_t(g, sceneT, w1T, b1c, w2T, b2c, blk=4096):
    grid = B // blk
    outT = pl.pallas_call(
        _mlp_body,
        grid=(grid,),
        in_specs=[
            pl.BlockSpec((blk, N_TAB * EMB), lambda i: (i, 0)),
            pl.BlockSpec((SCENE, blk), lambda i: (0, i)),
            pl.BlockSpec((HID, N_TAB * EMB + SCENE), lambda i: (0, 0)),
            pl.BlockSpec((HID, 1), lambda i: (0, 0)),
            pl.BlockSpec((OUT, HID), lambda i: (0, 0)),
            pl.BlockSpec((OUT, 1), lambda i: (0, 0)),
        ],
        out_specs=pl.BlockSpec((OUT, blk), lambda i: (0, i)),
        out_shape=jax.ShapeDtypeStruct((OUT, B), jnp.float32),
    )(g, sceneT, w1T, b1c, w2T, b2c)
    return outT


def kernel(killer_id, victim_id, move_id, stage_id, scene_tags,
           killer_table, victim_table, move_table, stage_table,
           W1, b1, W2, b2):
    ids_r = [i.reshape(NW, JPT, CHUNK) for i in
             (killer_id, victim_id, move_id, stage_id)]
    ctab = jnp.concatenate(
        [killer_table, victim_table, move_table, stage_table], axis=0
    )
    g = _sc_gather(*ids_r, ctab)
    outT = _mlp_t(g, scene_tags.T, W1.T, b1.reshape(HID, 1),
                  W2.T, b2.reshape(OUT, 1))
    return outT.T


# speedup vs baseline: 1.1577x; 1.0310x over previous
import functools

import jax
import jax.numpy as jnp
from jax import lax
from jax.experimental import pallas as pl
from jax.experimental.pallas import tpu as pltpu
from jax.experimental.pallas import tpu_sc as plsc

B = 16384
EMB = 32
N_TAB = 4
HID = 256
OUT = 20
SCENE = 20

NC, NS = 2, 16
NW = NC * NS
BPW = B // NW
CHUNK = 128
JPT = BPW // CHUNK


TAB_ROWS = 86 + 86 + 24 + 8
TAB_OFFS = (0, 86, 172, 196)


def _sc_gather(kid_r, vid_r, mid_r, sid_r, ctab):
    mesh = plsc.VectorSubcoreMesh(core_axis_name="c", subcore_axis_name="s")

    @functools.partial(
        pl.kernel,
        mesh=mesh,
        compiler_params=pltpu.CompilerParams(use_tc_tiling_on_sc=False),
        out_type=jax.ShapeDtypeStruct((B, N_TAB * EMB), jnp.float32),
        scratch_types=[
            pltpu.VMEM((N_TAB, JPT, CHUNK), jnp.int32),
            pltpu.VMEM((N_TAB * BPW, EMB), jnp.float32),
            pltpu.VMEM_SHARED((TAB_ROWS, EMB), jnp.float32),
            pltpu.SemaphoreType.DMA,
            pltpu.SemaphoreType.DMA,
            pltpu.SemaphoreType.DMA,
        ],
    )
    def k(kid_h, vid_h, mid_h, sid_h, ctab_h, out_h,
          idx_v, rows_v, ctab_s, sem_i, sem_g, sem_o):
        sid_ax = lax.axis_index("s")
        wid = sid_ax * NC + lax.axis_index("c")
        base = wid * BPW
        ids = (kid_h, vid_h, mid_h, sid_h)
        cps = [
            pltpu.async_copy(ids[t].at[wid], idx_v.at[t], sem_i)
            for t in range(N_TAB)
        ]
        @pl.when(sid_ax == 0)
        def _stage_tables():
            pltpu.sync_copy(ctab_h, ctab_s)
        for cp in cps:
            cp.wait()
        for t in range(1, N_TAB):
            for j in range(JPT):
                for m in range(CHUNK // 16):
                    sl = (t, j, pl.ds(16 * m, 16))
                    idx_v[sl] = idx_v[sl] + TAB_OFFS[t]
        plsc.subcore_barrier()
        cps = [
            [
                pltpu.async_copy(
                    ctab_s.at[idx_v.at[t, j]],
                    rows_v.at[pl.ds(BPW * t + CHUNK * j, CHUNK)],
                    sem_g,
                )
                for j in range(JPT)
            ]
            for t in range(N_TAB)
        ]
        wbs = []
        for t in range(N_TAB):
            for cp in cps[t]:
                cp.wait()
            wbs.append(
                pltpu.async_copy(
                    rows_v.at[pl.ds(BPW * t, BPW)],
                    out_h.at[pl.ds(base, BPW), pl.ds(EMB * t, EMB)],
                    sem_o,
                )
            )
        for cp in wbs:
            cp.wait()

    return k(kid_r, vid_r, mid_r, sid_r, ctab)


def _dot(a, b, dims):
    return lax.dot_general(a, b, dimension_numbers=(dims, ((), ())),
                           preferred_element_type=jnp.float32)


def _mlp_body(g_ref, scT_ref, w1T_ref, b1_ref, w2T_ref, b2_ref, o_ref):
    hT = _dot(w1T_ref[:, pl.ds(0, N_TAB * EMB)], g_ref[...], ((1,), (1,)))
    hT += _dot(w1T_ref[:, pl.ds(N_TAB * EMB, SCENE)], scT_ref[...], ((1,), (0,)))
    hT = jnp.maximum(hT + b1_ref[...], 0.0)
    o_ref[...] = _dot(w2T_ref[...], hT, ((1,), (0,))) + b2_ref[...]


def _mlp_t(g, sceneT, w1T, b1c, w2T, b2c, blk=4096):
    grid = B // blk
    outT = pl.pallas_call(
        _mlp_body,
        grid=(grid,),
        in_specs=[
            pl.BlockSpec((blk, N_TAB * EMB), lambda i: (i, 0)),
            pl.BlockSpec((SCENE, blk), lambda i: (0, i)),
            pl.BlockSpec((HID, N_TAB * EMB + SCENE), lambda i: (0, 0)),
            pl.BlockSpec((HID, 1), lambda i: (0, 0)),
            pl.BlockSpec((OUT, HID), lambda i: (0, 0)),
            pl.BlockSpec((OUT, 1), lambda i: (0, 0)),
        ],
        out_specs=pl.BlockSpec((OUT, blk), lambda i: (0, i)),
        out_shape=jax.ShapeDtypeStruct((OUT, B), jnp.float32),
    )(g, sceneT, w1T, b1c, w2T, b2c)
    return outT


def kernel(killer_id, victim_id, move_id, stage_id, scene_tags,
           killer_table, victim_table, move_table, stage_table,
           W1, b1, W2, b2):
    ids_r = [i.reshape(NW, JPT, CHUNK) for i in
             (killer_id, victim_id, move_id, stage_id)]
    ctab = jnp.concatenate(
        [killer_table, victim_table, move_table, stage_table], axis=0
    )
    g = _sc_gather(*ids_r, ctab)
    outT = _mlp_t(g, scene_tags.T, W1.T, b1.reshape(HID, 1),
                  W2.T, b2.reshape(OUT, 1))
    return outT.T
